# Initial kernel scaffold; baseline (speedup 1.0000x reference)
import functools

import jax
import jax.numpy as jnp
from jax import lax
from jax.experimental import pallas as pl
from jax.experimental.pallas import tpu as pltpu
from jax.experimental.pallas import tpu_sc as plsc

M = 2048
HIDDEN = 1024
INTER = 512
E = 64
CLAMP_LO = -10.0
CLAMP_HI = 10.0

TM = 64
NT = M // TM + E
P = NT * TM

NC = 2
NS = 16
NW = NC * NS
BPW = M // NW

_sc_mesh = plsc.VectorSubcoreMesh(core_axis_name="c", subcore_axis_name="s")


@functools.partial(
    pl.kernel,
    mesh=_sc_mesh,
    out_type=[
        jax.ShapeDtypeStruct((P, HIDDEN), jnp.float32),
        jax.ShapeDtypeStruct((P, 16), jnp.float32),
    ],
    scratch_types=[
        pltpu.VMEM((BPW,), jnp.int32),
        pltpu.VMEM((BPW, HIDDEN), jnp.float32),
        pltpu.VMEM((BPW, 16), jnp.float32),
        pltpu.SemaphoreType.DMA,
        pltpu.SemaphoreType.DMA,
    ],
)
def _sc_scatter(h_hbm, g_hbm, slot_hbm, px_hbm, pg_hbm,
                idx_v, rows_v, grows_v, sem_x, sem_g):
    wid = lax.axis_index("s") * NC + lax.axis_index("c")
    base = wid * BPW
    pltpu.sync_copy(slot_hbm.at[pl.ds(base, BPW)], idx_v)
    pltpu.sync_copy(h_hbm.at[pl.ds(base, BPW)], rows_v)
    pltpu.sync_copy(g_hbm.at[pl.ds(base, BPW)], grows_v)
    cp_x = pltpu.async_copy(rows_v, px_hbm.at[idx_v], sem_x)
    cp_g = pltpu.async_copy(grows_v, pg_hbm.at[idx_v], sem_g)
    cp_x.wait()
    cp_g.wait()


@functools.partial(
    pl.kernel,
    mesh=_sc_mesh,
    out_type=jax.ShapeDtypeStruct((M, HIDDEN), jnp.float32),
    scratch_types=[
        pltpu.VMEM((BPW,), jnp.int32),
        pltpu.VMEM((BPW, HIDDEN), jnp.float32),
        pltpu.SemaphoreType.DMA,
    ],
)
def _sc_gather(py_hbm, slot_hbm, out_hbm, idx_v, rows_v, sem):
    wid = lax.axis_index("s") * NC + lax.axis_index("c")
    base = wid * BPW
    pltpu.sync_copy(slot_hbm.at[pl.ds(base, BPW)], idx_v)
    pltpu.async_copy(py_hbm.at[idx_v], rows_v, sem).wait()
    pltpu.sync_copy(rows_v, out_hbm.at[pl.ds(base, BPW)])


def _gemm_body(tg_ref, xm_ref, act_ref,
               x_ref, gw_ref, uw_ref, dw_ref, pg_ref, y_ref):
    i = pl.program_id(0)

    @pl.when(act_ref[i] == 1)
    def _():
        x = x_ref[...]
        gw = gw_ref[0]
        uw = uw_ref[0]
        dn = (((1,), (1,)), ((), ()))
        g = lax.dot_general(x, gw, dn,
                            preferred_element_type=jnp.float32,
                            precision=lax.Precision.HIGHEST)
        g = jnp.minimum(g, CLAMP_HI)
        u = lax.dot_general(x, uw, dn,
                            preferred_element_type=jnp.float32,
                            precision=lax.Precision.HIGHEST)
        u = jnp.clip(u, CLAMP_LO, CLAMP_HI)
        sig = 1.0 / (1.0 + jnp.exp(-g))
        h = (g * sig) * u
        h = h * pg_ref[:, 0:1]
        dw = dw_ref[0]
        y = lax.dot_general(h, dw, dn,
                            preferred_element_type=jnp.float32,
                            precision=lax.Precision.HIGHEST)
        y_ref[...] = y


def _grouped_gemm(tg, xm, act, padded_x, gate_weight, up_weight,
                  down_weight, padded_g):
    grid_spec = pltpu.PrefetchScalarGridSpec(
        num_scalar_prefetch=3,
        grid=(NT,),
        in_specs=[
            pl.BlockSpec((TM, HIDDEN), lambda i, tg, xm, act: (xm[i], 0)),
            pl.BlockSpec((1, INTER, HIDDEN),
                         lambda i, tg, xm, act: (tg[i], 0, 0)),
            pl.BlockSpec((1, INTER, HIDDEN),
                         lambda i, tg, xm, act: (tg[i], 0, 0)),
            pl.BlockSpec((1, HIDDEN, INTER),
                         lambda i, tg, xm, act: (tg[i], 0, 0)),
            pl.BlockSpec((TM, 16), lambda i, tg, xm, act: (xm[i], 0)),
        ],
        out_specs=pl.BlockSpec((TM, HIDDEN),
                               lambda i, tg, xm, act: (xm[i], 0)),
    )
    return pl.pallas_call(
        _gemm_body,
        grid_spec=grid_spec,
        out_shape=jax.ShapeDtypeStruct((P, HIDDEN), jnp.float32),
    )(tg, xm, act, padded_x, gate_weight, up_weight, down_weight, padded_g)


def _routing_metadata(e):
    onehot = (e[:, None] == jnp.arange(E, dtype=jnp.int32)[None, :])
    onehot = onehot.astype(jnp.int32)
    incl = jnp.cumsum(onehot, axis=0)
    rank = jnp.sum(incl * onehot, axis=1) - 1
    counts = incl[-1]
    tiles = (counts + TM - 1) // TM
    tile_cum = jnp.cumsum(tiles)
    tile_start = tile_cum - tiles
    total = tile_cum[-1]
    slot = TM * jnp.sum(onehot * tile_start[None, :], axis=1) + rank
    ii = jnp.arange(NT, dtype=jnp.int32)
    tg0 = jnp.sum((tile_cum[None, :] <= ii[:, None]).astype(jnp.int32),
                  axis=1)
    lastg = jnp.take(tg0, total - 1)
    tg = jnp.where(ii < total, tg0, lastg).astype(jnp.int32)
    xm = jnp.where(ii < total, ii, total - 1).astype(jnp.int32)
    act = (ii < total).astype(jnp.int32)
    return slot.astype(jnp.int32), tg, xm, act


def kernel(flat_h, flat_idx, flat_gate, gate_weight, up_weight, down_weight):
    e = flat_idx[:, 0].astype(jnp.int32)
    slot, tg, xm, act = _routing_metadata(e)
    gate16 = jnp.broadcast_to(flat_gate.astype(jnp.float32), (M, 16))

    padded_x, padded_g = _sc_scatter(flat_h, gate16, slot)
    padded_y = _grouped_gemm(tg, xm, act, padded_x, gate_weight, up_weight,
                             down_weight, padded_g)
    return _sc_gather(padded_y, slot)


# speedup vs baseline: 3.5968x; 3.5968x over previous
import functools

import jax
import jax.numpy as jnp
from jax import lax
from jax.experimental import pallas as pl
from jax.experimental.pallas import tpu as pltpu
from jax.experimental.pallas import tpu_sc as plsc

M = 2048
HIDDEN = 1024
INTER = 512
E = 64
CLAMP_LO = -10.0
CLAMP_HI = 10.0

TM = 64
NT = M // TM + E
P = NT * TM
GW = 128

NC = 2
NS = 16
NW = NC * NS
BPW = M // NW

@functools.lru_cache(maxsize=None)
def _sc_scatter_kernel():
    mesh = plsc.VectorSubcoreMesh(core_axis_name="c", subcore_axis_name="s")

    @functools.partial(
        pl.kernel,
        mesh=mesh,
        out_type=[
            jax.ShapeDtypeStruct((P, HIDDEN), jnp.float32),
            jax.ShapeDtypeStruct((P, GW), jnp.float32),
        ],
        scratch_types=[
            pltpu.VMEM((BPW,), jnp.int32),
            pltpu.VMEM((BPW, HIDDEN), jnp.float32),
            pltpu.VMEM((BPW, GW), jnp.float32),
            pltpu.SemaphoreType.DMA,
            pltpu.SemaphoreType.DMA,
        ],
    )
    def _sc_scatter(h_hbm, g_hbm, slot_hbm, px_hbm, pg_hbm,
                    idx_v, rows_v, grows_v, sem_x, sem_g):
        wid = lax.axis_index("s") * NC + lax.axis_index("c")
        base = wid * BPW
        pltpu.sync_copy(slot_hbm.at[pl.ds(base, BPW)], idx_v)
        pltpu.sync_copy(h_hbm.at[pl.ds(base, BPW)], rows_v)
        pltpu.sync_copy(g_hbm.at[pl.ds(base, BPW)], grows_v)
        cp_x = pltpu.async_copy(rows_v, px_hbm.at[idx_v], sem_x)
        cp_g = pltpu.async_copy(grows_v, pg_hbm.at[idx_v], sem_g)
        cp_x.wait()
        cp_g.wait()

    return _sc_scatter


@functools.lru_cache(maxsize=None)
def _sc_gather_kernel():
    mesh = plsc.VectorSubcoreMesh(core_axis_name="c", subcore_axis_name="s")

    @functools.partial(
        pl.kernel,
        mesh=mesh,
        out_type=jax.ShapeDtypeStruct((M, HIDDEN), jnp.float32),
        scratch_types=[
            pltpu.VMEM((BPW,), jnp.int32),
            pltpu.VMEM((BPW, HIDDEN), jnp.float32),
            pltpu.SemaphoreType.DMA,
        ],
    )
    def _sc_gather(py_hbm, slot_hbm, out_hbm, idx_v, rows_v, sem):
        wid = lax.axis_index("s") * NC + lax.axis_index("c")
        base = wid * BPW
        pltpu.sync_copy(slot_hbm.at[pl.ds(base, BPW)], idx_v)
        pltpu.async_copy(py_hbm.at[idx_v], rows_v, sem).wait()
        pltpu.sync_copy(rows_v, out_hbm.at[pl.ds(base, BPW)])

    return _sc_gather


def _gemm_body(tg_ref, xm_ref, act_ref,
               x_ref, gw_ref, uw_ref, dw_ref, pg_ref, y_ref):
    i = pl.program_id(0)

    @pl.when(act_ref[i] == 1)
    def _():
        x = x_ref[...]
        gw = gw_ref[0]
        uw = uw_ref[0]
        dn = (((1,), (1,)), ((), ()))
        g = lax.dot_general(x, gw, dn,
                            preferred_element_type=jnp.float32,
                            precision=lax.Precision.HIGHEST)
        g = jnp.minimum(g, CLAMP_HI)
        u = lax.dot_general(x, uw, dn,
                            preferred_element_type=jnp.float32,
                            precision=lax.Precision.HIGHEST)
        u = jnp.clip(u, CLAMP_LO, CLAMP_HI)
        sig = 1.0 / (1.0 + jnp.exp(-g))
        h = (g * sig) * u
        h = h * pg_ref[:, 0:1]
        dw = dw_ref[0]
        y = lax.dot_general(h, dw, dn,
                            preferred_element_type=jnp.float32,
                            precision=lax.Precision.HIGHEST)
        y_ref[...] = y


def _grouped_gemm(tg, xm, act, padded_x, gate_weight, up_weight,
                  down_weight, padded_g):
    grid_spec = pltpu.PrefetchScalarGridSpec(
        num_scalar_prefetch=3,
        grid=(NT,),
        in_specs=[
            pl.BlockSpec((TM, HIDDEN), lambda i, tg, xm, act: (xm[i], 0)),
            pl.BlockSpec((1, INTER, HIDDEN),
                         lambda i, tg, xm, act: (tg[i], 0, 0)),
            pl.BlockSpec((1, INTER, HIDDEN),
                         lambda i, tg, xm, act: (tg[i], 0, 0)),
            pl.BlockSpec((1, HIDDEN, INTER),
                         lambda i, tg, xm, act: (tg[i], 0, 0)),
            pl.BlockSpec((TM, GW), lambda i, tg, xm, act: (xm[i], 0)),
        ],
        out_specs=pl.BlockSpec((TM, HIDDEN),
                               lambda i, tg, xm, act: (xm[i], 0)),
    )
    return pl.pallas_call(
        _gemm_body,
        grid_spec=grid_spec,
        out_shape=jax.ShapeDtypeStruct((P, HIDDEN), jnp.float32),
    )(tg, xm, act, padded_x, gate_weight, up_weight, down_weight, padded_g)


def _routing_metadata(e):
    onehot = (e[:, None] == jnp.arange(E, dtype=jnp.int32)[None, :])
    onehot = onehot.astype(jnp.int32)
    incl = jnp.cumsum(onehot, axis=0)
    rank = jnp.sum(incl * onehot, axis=1) - 1
    counts = incl[-1]
    tiles = (counts + TM - 1) // TM
    tile_cum = jnp.cumsum(tiles)
    tile_start = tile_cum - tiles
    total = tile_cum[-1]
    slot = TM * jnp.sum(onehot * tile_start[None, :], axis=1) + rank
    ii = jnp.arange(NT, dtype=jnp.int32)
    tg0 = jnp.sum((tile_cum[None, :] <= ii[:, None]).astype(jnp.int32),
                  axis=1)
    lastg = jnp.take(tg0, total - 1)
    tg = jnp.where(ii < total, tg0, lastg).astype(jnp.int32)
    xm = jnp.where(ii < total, ii, total - 1).astype(jnp.int32)
    act = (ii < total).astype(jnp.int32)
    return slot.astype(jnp.int32), tg, xm, act


def kernel(flat_h, flat_idx, flat_gate, gate_weight, up_weight, down_weight):
    e = flat_idx[:, 0].astype(jnp.int32)
    slot, tg, xm, act = _routing_metadata(e)
    gate16 = jnp.broadcast_to(flat_gate.astype(jnp.float32), (M, GW))

    padded_x, padded_g = _sc_scatter_kernel()(flat_h, gate16, slot)
    padded_y = _grouped_gemm(tg, xm, act, padded_x, gate_weight, up_weight,
                             down_weight, padded_g)
    return _sc_gather_kernel()(padded_y, slot)


# speedup vs baseline: 8.0662x; 2.2426x over previous
import functools

import jax
import jax.numpy as jnp
from jax import lax
from jax.experimental import pallas as pl
from jax.experimental.pallas import tpu as pltpu
from jax.experimental.pallas import tpu_sc as plsc

M = 2048
HIDDEN = 1024
INTER = 512
E = 64
CLAMP_LO = -10.0
CLAMP_HI = 10.0

TM = 64
NT = M // TM + E
P = NT * TM
GW = 128

NC = 2
NS = 16
NW = NC * NS
BPW = M // NW

@functools.lru_cache(maxsize=None)
def _sc_scatter_kernel():
    mesh = plsc.VectorSubcoreMesh(core_axis_name="c", subcore_axis_name="s")

    @functools.partial(
        pl.kernel,
        mesh=mesh,
        out_type=[
            jax.ShapeDtypeStruct((P, HIDDEN), jnp.float32),
            jax.ShapeDtypeStruct((P, GW), jnp.float32),
        ],
        scratch_types=[
            pltpu.VMEM((BPW,), jnp.int32),
            pltpu.VMEM((BPW, HIDDEN), jnp.float32),
            pltpu.VMEM((BPW, GW), jnp.float32),
            pltpu.SemaphoreType.DMA,
            pltpu.SemaphoreType.DMA,
        ],
    )
    def _sc_scatter(h_hbm, g_hbm, slot_hbm, px_hbm, pg_hbm,
                    idx_v, rows_v, grows_v, sem_x, sem_g):
        wid = lax.axis_index("s") * NC + lax.axis_index("c")
        base = wid * BPW
        pltpu.sync_copy(slot_hbm.at[pl.ds(base, BPW)], idx_v)
        pltpu.sync_copy(h_hbm.at[pl.ds(base, BPW)], rows_v)
        pltpu.sync_copy(g_hbm.at[pl.ds(base, BPW)], grows_v)
        cp_x = pltpu.async_copy(rows_v, px_hbm.at[idx_v], sem_x)
        cp_g = pltpu.async_copy(grows_v, pg_hbm.at[idx_v], sem_g)
        cp_x.wait()
        cp_g.wait()

    return _sc_scatter


@functools.lru_cache(maxsize=None)
def _sc_gather_kernel():
    mesh = plsc.VectorSubcoreMesh(core_axis_name="c", subcore_axis_name="s")

    @functools.partial(
        pl.kernel,
        mesh=mesh,
        out_type=jax.ShapeDtypeStruct((M, HIDDEN), jnp.float32),
        scratch_types=[
            pltpu.VMEM((BPW,), jnp.int32),
            pltpu.VMEM((BPW, HIDDEN), jnp.float32),
            pltpu.SemaphoreType.DMA,
        ],
    )
    def _sc_gather(py_hbm, slot_hbm, out_hbm, idx_v, rows_v, sem):
        wid = lax.axis_index("s") * NC + lax.axis_index("c")
        base = wid * BPW
        pltpu.sync_copy(slot_hbm.at[pl.ds(base, BPW)], idx_v)
        pltpu.async_copy(py_hbm.at[idx_v], rows_v, sem).wait()
        pltpu.sync_copy(rows_v, out_hbm.at[pl.ds(base, BPW)])

    return _sc_gather


def _gemm_body(tg_ref, xm_ref, act_ref,
               x_ref, gw_ref, uw_ref, dw_ref, pg_ref, y_ref):
    i = pl.program_id(0)

    @pl.when(act_ref[i] == 1)
    def _():
        x = x_ref[...]
        gw = gw_ref[0]
        uw = uw_ref[0]
        dn = (((1,), (1,)), ((), ()))
        g = lax.dot_general(x, gw, dn,
                            preferred_element_type=jnp.float32,
                            precision=lax.Precision.DEFAULT)
        g = jnp.minimum(g, CLAMP_HI)
        u = lax.dot_general(x, uw, dn,
                            preferred_element_type=jnp.float32,
                            precision=lax.Precision.DEFAULT)
        u = jnp.clip(u, CLAMP_LO, CLAMP_HI)
        sig = 1.0 / (1.0 + jnp.exp(-g))
        h = (g * sig) * u
        h = h * pg_ref[:, 0:1]
        dw = dw_ref[0]
        y = lax.dot_general(h, dw, dn,
                            preferred_element_type=jnp.float32,
                            precision=lax.Precision.DEFAULT)
        y_ref[...] = y


def _grouped_gemm(tg, xm, act, padded_x, gate_weight, up_weight,
                  down_weight, padded_g):
    grid_spec = pltpu.PrefetchScalarGridSpec(
        num_scalar_prefetch=3,
        grid=(NT,),
        in_specs=[
            pl.BlockSpec((TM, HIDDEN), lambda i, tg, xm, act: (xm[i], 0)),
            pl.BlockSpec((1, INTER, HIDDEN),
                         lambda i, tg, xm, act: (tg[i], 0, 0)),
            pl.BlockSpec((1, INTER, HIDDEN),
                         lambda i, tg, xm, act: (tg[i], 0, 0)),
            pl.BlockSpec((1, HIDDEN, INTER),
                         lambda i, tg, xm, act: (tg[i], 0, 0)),
            pl.BlockSpec((TM, GW), lambda i, tg, xm, act: (xm[i], 0)),
        ],
        out_specs=pl.BlockSpec((TM, HIDDEN),
                               lambda i, tg, xm, act: (xm[i], 0)),
    )
    return pl.pallas_call(
        _gemm_body,
        grid_spec=grid_spec,
        out_shape=jax.ShapeDtypeStruct((P, HIDDEN), jnp.float32),
    )(tg, xm, act, padded_x, gate_weight, up_weight, down_weight, padded_g)


def _routing_metadata(e):
    onehot = (e[:, None] == jnp.arange(E, dtype=jnp.int32)[None, :])
    onehot = onehot.astype(jnp.int32)
    incl = jnp.cumsum(onehot, axis=0)
    rank = jnp.sum(incl * onehot, axis=1) - 1
    counts = incl[-1]
    tiles = (counts + TM - 1) // TM
    tile_cum = jnp.cumsum(tiles)
    tile_start = tile_cum - tiles
    total = tile_cum[-1]
    slot = TM * jnp.sum(onehot * tile_start[None, :], axis=1) + rank
    ii = jnp.arange(NT, dtype=jnp.int32)
    tg0 = jnp.sum((tile_cum[None, :] <= ii[:, None]).astype(jnp.int32),
                  axis=1)
    lastg = jnp.take(tg0, total - 1)
    tg = jnp.where(ii < total, tg0, lastg).astype(jnp.int32)
    xm = jnp.where(ii < total, ii, total - 1).astype(jnp.int32)
    act = (ii < total).astype(jnp.int32)
    return slot.astype(jnp.int32), tg, xm, act


def kernel(flat_h, flat_idx, flat_gate, gate_weight, up_weight, down_weight):
    e = flat_idx[:, 0].astype(jnp.int32)
    slot, tg, xm, act = _routing_metadata(e)
    gate16 = jnp.broadcast_to(flat_gate.astype(jnp.float32), (M, GW))

    padded_x, padded_g = _sc_scatter_kernel()(flat_h, gate16, slot)
    padded_y = _grouped_gemm(tg, xm, act, padded_x, gate_weight, up_weight,
                             down_weight, padded_g)
    return _sc_gather_kernel()(padded_y, slot)
